# baseline (speedup 1.0000x reference)
import functools

import jax
import jax.numpy as jnp
from jax.experimental import pallas as pl


def _top3_network(d0, nb, m):
    inf = jnp.float32(jnp.inf)
    nchunk = m // 128
    if m % 128 == 0 and nchunk >= 4 and (nchunk & (nchunk - 1)) == 0:
        chunks = [(d0[:, j * 128:(j + 1) * 128], None, None)
                  for j in range(m // 128)]

        def merge(a, b):
            x1, x2, x3 = a
            y1, y2, y3 = b
            z1 = jnp.minimum(x1, y1)
            p = jnp.maximum(x1, y1)
            if x2 is None and y2 is None:
                return (z1, p, None)
            if x3 is None and y3 is None:
                q = jnp.minimum(x2, y2)
                h = jnp.maximum(x2, y2)
                z2 = jnp.minimum(p, q)
                z3 = jnp.minimum(jnp.maximum(p, q), h)
                return (z1, z2, z3)
            q = jnp.minimum(x2, y2)
            z2 = jnp.minimum(p, q)
            t = jnp.maximum(p, q)
            s = jnp.minimum(x3, y3)
            z3 = jnp.minimum(t, s)
            return (z1, z2, z3)

        while len(chunks) > 1:
            chunks = [merge(chunks[i], chunks[i + 1])
                      for i in range(0, len(chunks), 2)]
        c1, c2, c3 = chunks[0]
        m1 = jnp.min(c1, axis=1, keepdims=True)
        e = c1 == m1
        c1 = jnp.where(e, c2, c1)
        c2 = jnp.where(e, c3, c2)
        m2 = jnp.min(c1, axis=1, keepdims=True)
        e = c1 == m2
        c1 = jnp.where(e, c2, c1)
        m3 = jnp.min(c1, axis=1, keepdims=True)
        return m1, m2, m3
    m1 = jnp.min(d0, axis=1, keepdims=True)
    dm = jnp.where(d0 == m1, inf, d0)
    m2 = jnp.min(dm, axis=1, keepdims=True)
    dm = jnp.where(dm == m2, inf, dm)
    m3 = jnp.min(dm, axis=1, keepdims=True)
    return m1, m2, m3


def _fp_block_kernel(u_ref, rhs_ref, uf_ref, kf_ref,
                     w1a_ref, w1b_ref, b1_ref, w2_ref, b2_ref, o_ref,
                     *, nb: int, m: int):
    zero = jnp.float32(0.0)
    inf = jnp.float32(jnp.inf)
    bf16, f32 = jnp.bfloat16, jnp.float32
    u = u_ref[0]
    n2u = -(u + u)
    uh = n2u.astype(bf16)
    r = n2u - uh.astype(f32)
    um = r.astype(bf16)
    ul = (r - um.astype(f32)).astype(bf16)
    ones3 = jnp.ones((nb, 3), dtype=bf16)
    lhs = jnp.concatenate([uh, uh, um, uh, ul, um, ones3], axis=1)
    unorm = jnp.sum(u * u, axis=1, keepdims=True)
    eps = jnp.float32(1e-10)
    d0 = jnp.maximum(
        jnp.dot(lhs, rhs_ref[0], preferred_element_type=jnp.float32)
        + unorm,
        eps)

    m1, m2, m3 = _top3_network(d0, nb, m)

    r1 = 1.0 / m1
    r2 = 1.0 / m2
    r3 = 1.0 / m3
    rnorm = 1.0 / (r1 + r2 + r3)
    sel = jnp.where(d0 <= m3, d0, inf)
    wmat = (1.0 / sel).astype(bf16)

    interp = jnp.dot(wmat, kf_ref[0], preferred_element_type=jnp.float32)
    interp = interp * rnorm
    h = jnp.dot(interp, w1a_ref[...], preferred_element_type=jnp.float32)
    h = h + jnp.dot(uf_ref[0], w1b_ref[...], preferred_element_type=jnp.float32)
    h = jnp.maximum(h + b1_ref[...], zero)
    o = jnp.dot(h, w2_ref[...], preferred_element_type=jnp.float32)
    o_ref[0] = jnp.maximum(o + b2_ref[...], zero)


def _split3(x):
    bf16 = jnp.bfloat16
    h32 = jax.lax.reduce_precision(x, exponent_bits=8, mantissa_bits=7)
    r = x - h32
    m32 = jax.lax.reduce_precision(r, exponent_bits=8, mantissa_bits=7)
    l = (r - m32).astype(bf16)
    return h32.astype(bf16), m32.astype(bf16), l


def kernel(unknown, known, unknow_feats, known_feats, W1, b1, W2, b2):
    B, n, _ = unknown.shape
    m = known.shape[1]
    C1 = unknow_feats.shape[2]
    C2 = known_feats.shape[2]
    nb = min(4096, n)
    bf16 = jnp.bfloat16

    kt = jnp.transpose(known, (0, 2, 1))
    kh, km, kl = _split3(kt)
    knorm = jnp.sum(known * known, axis=2, keepdims=True)
    knorm = jnp.transpose(knorm, (0, 2, 1))
    gh, gm, gl = _split3(knorm)
    rhs = jnp.concatenate([kh, km, kh, kl, kh, km, gh, gm, gl],
                          axis=1)

    w1a = jnp.transpose(W1[:, :C2])
    w1b = jnp.transpose(W1[:, C2:])
    w2t = jnp.transpose(W2)
    b1r = b1.reshape(1, -1)
    b2r = b2.reshape(1, -1)

    grid = (B, n // nb)
    out = pl.pallas_call(
        functools.partial(_fp_block_kernel, nb=nb, m=m),
        grid=grid,
        in_specs=[
            pl.BlockSpec((1, nb, 3), lambda b, i: (b, i, 0)),
            pl.BlockSpec((1, 21, m), lambda b, i: (b, 0, 0)),
            pl.BlockSpec((1, nb, C1), lambda b, i: (b, i, 0)),
            pl.BlockSpec((1, m, C2), lambda b, i: (b, 0, 0)),
            pl.BlockSpec((C2, 256), lambda b, i: (0, 0)),
            pl.BlockSpec((C1, 256), lambda b, i: (0, 0)),
            pl.BlockSpec((1, 256), lambda b, i: (0, 0)),
            pl.BlockSpec((256, 128), lambda b, i: (0, 0)),
            pl.BlockSpec((1, 128), lambda b, i: (0, 0)),
        ],
        out_specs=pl.BlockSpec((1, nb, 128), lambda b, i: (b, i, 0)),
        out_shape=jax.ShapeDtypeStruct((B, n, 128), jnp.float32),
    )(unknown, rhs, unknow_feats, known_feats.astype(bf16),
      w1a, w1b, b1r, w2t, b2r)
    return out


# speedup vs baseline: 1.0517x; 1.0517x over previous
import functools

import jax
import jax.numpy as jnp
from jax.experimental import pallas as pl


def _top3_network(d0, nb, m):
    inf = jnp.float32(jnp.inf)
    nchunk = m // 128
    if m % 128 == 0 and nchunk >= 4 and (nchunk & (nchunk - 1)) == 0:
        chunks = [(d0[:, j * 128:(j + 1) * 128], None, None)
                  for j in range(m // 128)]

        def merge(a, b):
            x1, x2, x3 = a
            y1, y2, y3 = b
            z1 = jnp.minimum(x1, y1)
            p = jnp.maximum(x1, y1)
            if x2 is None and y2 is None:
                return (z1, p, None)
            if x3 is None and y3 is None:
                q = jnp.minimum(x2, y2)
                h = jnp.maximum(x2, y2)
                z2 = jnp.minimum(p, q)
                z3 = jnp.minimum(jnp.maximum(p, q), h)
                return (z1, z2, z3)
            q = jnp.minimum(x2, y2)
            z2 = jnp.minimum(p, q)
            t = jnp.maximum(p, q)
            s = jnp.minimum(x3, y3)
            z3 = jnp.minimum(t, s)
            return (z1, z2, z3)

        while len(chunks) > 1:
            chunks = [merge(chunks[i], chunks[i + 1])
                      for i in range(0, len(chunks), 2)]
        c1, c2, c3 = chunks[0]
        m1 = jnp.min(c1, axis=1, keepdims=True)
        e = c1 == m1
        c1 = jnp.where(e, c2, c1)
        c2 = jnp.where(e, c3, c2)
        m2 = jnp.min(c1, axis=1, keepdims=True)
        e = c1 == m2
        c1 = jnp.where(e, c2, c1)
        m3 = jnp.min(c1, axis=1, keepdims=True)
        return m1, m2, m3
    m1 = jnp.min(d0, axis=1, keepdims=True)
    dm = jnp.where(d0 == m1, inf, d0)
    m2 = jnp.min(dm, axis=1, keepdims=True)
    dm = jnp.where(dm == m2, inf, dm)
    m3 = jnp.min(dm, axis=1, keepdims=True)
    return m1, m2, m3


def _fp_block_kernel(u_ref, rhs_ref, uf_ref, kf_ref,
                     w1a_ref, w1b_ref, b1_ref, w2_ref, b2_ref, o_ref,
                     *, nb: int, m: int):
    zero = jnp.float32(0.0)
    inf = jnp.float32(jnp.inf)
    bf16, f32 = jnp.bfloat16, jnp.float32
    u = u_ref[0]
    n2u = -(u + u)
    uh = n2u.astype(bf16)
    r = n2u - uh.astype(f32)
    um = r.astype(bf16)
    ul = (r - um.astype(f32)).astype(bf16)
    ones3 = jnp.ones((nb, 3), dtype=bf16)
    lhs = jnp.concatenate([uh, uh, um, uh, ul, um, ones3], axis=1)
    unorm = jnp.sum(u * u, axis=1, keepdims=True)
    eps = jnp.float32(1e-10)
    d0 = jnp.maximum(
        jnp.dot(lhs, rhs_ref[0], preferred_element_type=jnp.float32)
        + unorm,
        eps)

    m1, m2, m3 = _top3_network(d0, nb, m)

    r1 = 1.0 / m1
    r2 = 1.0 / m2
    r3 = 1.0 / m3
    rnorm = 1.0 / (r1 + r2 + r3)
    sel = jnp.where(d0 <= m3, d0, inf)
    wmat = 1.0 / sel

    interp = jnp.dot(wmat, kf_ref[0], preferred_element_type=jnp.float32)
    interp = interp * rnorm
    h = jnp.dot(interp, w1a_ref[...], preferred_element_type=jnp.float32)
    h = h + jnp.dot(uf_ref[0], w1b_ref[...], preferred_element_type=jnp.float32)
    h = jnp.maximum(h + b1_ref[...], zero)
    o = jnp.dot(h, w2_ref[...], preferred_element_type=jnp.float32)
    o_ref[0] = jnp.maximum(o + b2_ref[...], zero)


def _split3(x):
    bf16 = jnp.bfloat16
    h32 = jax.lax.reduce_precision(x, exponent_bits=8, mantissa_bits=7)
    r = x - h32
    m32 = jax.lax.reduce_precision(r, exponent_bits=8, mantissa_bits=7)
    l = (r - m32).astype(bf16)
    return h32.astype(bf16), m32.astype(bf16), l


def kernel(unknown, known, unknow_feats, known_feats, W1, b1, W2, b2):
    B, n, _ = unknown.shape
    m = known.shape[1]
    C1 = unknow_feats.shape[2]
    C2 = known_feats.shape[2]
    nb = min(4096, n)
    bf16 = jnp.bfloat16

    kt = jnp.transpose(known, (0, 2, 1))
    kh, km, kl = _split3(kt)
    knorm = jnp.sum(known * known, axis=2, keepdims=True)
    knorm = jnp.transpose(knorm, (0, 2, 1))
    gh, gm, gl = _split3(knorm)
    rhs = jnp.concatenate([kh, km, kh, kl, kh, km, gh, gm, gl],
                          axis=1)

    w1a = jnp.transpose(W1[:, :C2])
    w1b = jnp.transpose(W1[:, C2:])
    w2t = jnp.transpose(W2)
    b1r = b1.reshape(1, -1)
    b2r = b2.reshape(1, -1)

    grid = (B, n // nb)
    out = pl.pallas_call(
        functools.partial(_fp_block_kernel, nb=nb, m=m),
        grid=grid,
        in_specs=[
            pl.BlockSpec((1, nb, 3), lambda b, i: (b, i, 0)),
            pl.BlockSpec((1, 21, m), lambda b, i: (b, 0, 0)),
            pl.BlockSpec((1, nb, C1), lambda b, i: (b, i, 0)),
            pl.BlockSpec((1, m, C2), lambda b, i: (b, 0, 0)),
            pl.BlockSpec((C2, 256), lambda b, i: (0, 0)),
            pl.BlockSpec((C1, 256), lambda b, i: (0, 0)),
            pl.BlockSpec((1, 256), lambda b, i: (0, 0)),
            pl.BlockSpec((256, 128), lambda b, i: (0, 0)),
            pl.BlockSpec((1, 128), lambda b, i: (0, 0)),
        ],
        out_specs=pl.BlockSpec((1, nb, 128), lambda b, i: (b, i, 0)),
        out_shape=jax.ShapeDtypeStruct((B, n, 128), jnp.float32),
    )(unknown, rhs, unknow_feats, known_feats, w1a, w1b, b1r, w2t, b2r)
    return out
